# Initial kernel scaffold; baseline (speedup 1.0000x reference)
import jax
import jax.numpy as jnp
from jax.experimental import pallas as pl


def kernel(x, batch, Wg, bg, W, b):
    raise NotImplementedError("write your pallas kernel here")


# speedup vs baseline: 1.6882x; 1.6882x over previous
import functools

import jax
import jax.numpy as jnp
from jax.experimental import pallas as pl
from jax.experimental.pallas import tpu as pltpu

N = 50000
D = 256
G = 512
BLK = 512


def _body(x_ref, batch_ref, wg_ref, bg_ref, w_ref, b_ref, out_ref,
          s_acc, c_acc, *, nblk):
    i = pl.program_id(0)

    @pl.when(i == 0)
    def _init():
        s_acc[...] = jnp.zeros_like(s_acc)
        c_acc[...] = jnp.zeros_like(c_acc)

    xb = x_ref[...]
    ids = batch_ref[0]
    z = jax.lax.dot_general(
        xb, wg_ref[...], (((1,), (0,)), ((), ())),
        preferred_element_type=jnp.float32,
        precision=jax.lax.Precision.HIGHEST)
    z = z + bg_ref[0, 0]
    gate = 1.0 / (1.0 + jnp.exp(-z))
    y = gate * xb
    seg = jax.lax.broadcasted_iota(jnp.int32, (BLK, G), 1)
    p = (ids.reshape(BLK, 1) == seg).astype(jnp.float32)
    s_acc[...] += jax.lax.dot_general(
        p, y, (((0,), (0,)), ((), ())),
        preferred_element_type=jnp.float32,
        precision=jax.lax.Precision.HIGHEST)
    c_acc[...] += jnp.sum(p * gate, axis=0, keepdims=True)

    @pl.when(i == nblk - 1)
    def _fin():
        out_ref[...] = jax.lax.dot_general(
            s_acc[...], w_ref[...], (((1,), (0,)), ((), ())),
            preferred_element_type=jnp.float32,
            precision=jax.lax.Precision.HIGHEST) \
            + c_acc[...].reshape(G, 1) * b_ref[...].reshape(1, D)


def kernel(x, batch, Wg, bg, W, b):
    n_pad = ((N + BLK - 1) // BLK) * BLK
    nblk = n_pad // BLK
    xp = jnp.pad(x, ((0, n_pad - N), (0, 0)))
    bp = jnp.pad(batch.astype(jnp.int32), (0, n_pad - N),
                 constant_values=G)
    bp3 = bp.reshape(nblk, 1, BLK)

    out = pl.pallas_call(
        functools.partial(_body, nblk=nblk),
        grid=(nblk,),
        in_specs=[
            pl.BlockSpec((BLK, D), lambda i: (i, 0)),
            pl.BlockSpec((1, 1, BLK), lambda i: (i, 0, 0)),
            pl.BlockSpec((D, 1), lambda i: (0, 0)),
            pl.BlockSpec((1, 1), lambda i: (0, 0)),
            pl.BlockSpec((D, D), lambda i: (0, 0)),
            pl.BlockSpec((1, D), lambda i: (0, 0)),
        ],
        out_specs=pl.BlockSpec((G, D), lambda i: (0, 0)),
        out_shape=jax.ShapeDtypeStruct((G, D), jnp.float32),
        scratch_shapes=[
            pltpu.VMEM((G, D), jnp.float32),
            pltpu.VMEM((1, G), jnp.float32),
        ],
    )(xp, bp3, Wg, bg.reshape(1, 1), W, b.reshape(1, D))
    return out


# speedup vs baseline: 2.5021x; 1.4821x over previous
import functools

import jax
import jax.numpy as jnp
from jax import lax
from jax.experimental import pallas as pl
from jax.experimental.pallas import tpu as pltpu
from jax.experimental.pallas import tpu_sc as plsc

N = 50000
D = 256
G = 512
L = 16
NC = 2
NS = 16
NW = NC * NS
SPW = G // NW
C = 128
DK = D // L
DL = D + L
NROW16 = N // L


def _masked_pick(vec, iot, sel):
    selv = lax.broadcast(sel, (L,))
    return jnp.sum(jnp.where(iot == selv, vec, 0))


def _make_sc_kernel():
    mesh = plsc.VectorSubcoreMesh(core_axis_name="c", subcore_axis_name="s")

    @functools.partial(
        pl.kernel,
        out_type=jax.ShapeDtypeStruct((G, DL), jnp.float32),
        mesh=mesh,
        compiler_params=pltpu.CompilerParams(needs_layout_passes=False),
        scratch_types=[
            pltpu.VMEM((C, D), jnp.float32),
            pltpu.VMEM((C // L, L), jnp.int32),
            pltpu.VMEM((SPW, DL), jnp.float32),
            pltpu.VMEM((D,), jnp.float32),
            pltpu.VMEM((L,), jnp.float32),
            pltpu.VMEM((48,), jnp.int32),
        ],
    )
    def sc_kernel(x_hbm, ids2_hbm, wg_hbm, bg_hbm, bounds_hbm,
                  s_out,
                  x_v, ids_v, acc_v, wg_v, bg_v, bnd_v):
        cid = lax.axis_index("c")
        sid = lax.axis_index("s")
        wid = sid * NC + cid
        seg0 = pl.multiple_of(wid * SPW, SPW)

        pltpu.sync_copy(wg_hbm, wg_v)
        pltpu.sync_copy(bg_hbm, bg_v)
        pltpu.sync_copy(bounds_hbm, bnd_v)

        zeros16 = jnp.zeros((L,), jnp.float32)
        for i in range(SPW):
            for k in range(DK + 1):
                acc_v[i, pl.ds(L * k, L)] = zeros16

        iot = lax.iota(jnp.int32, L)
        b0 = bnd_v[pl.ds(0, L)]
        b1 = bnd_v[pl.ds(L, L)]
        b2 = bnd_v[pl.ds(2 * L, L)]
        lo = _masked_pick(b0, iot, wid) + _masked_pick(b1, iot, wid - L)
        hi = (_masked_pick(b0, iot, wid + 1)
              + _masked_pick(b1, iot, wid + 1 - L)
              + _masked_pick(b2, iot, wid + 1 - 2 * L))

        wgk = [wg_v[pl.ds(L * k, L)] for k in range(DK)]
        bg16 = bg_v[...]
        lane0 = (iot == 0).astype(jnp.float32)

        jlo = lo // C
        jhi = (hi + C - 1) // C

        def chunk_body(j, done):
            cb = pl.multiple_of(j * C, C)
            cbx = pl.multiple_of(jnp.minimum(cb, N - C), L)
            shift = cb - cbx
            pltpu.sync_copy(x_hbm.at[pl.ds(cbx, C)], x_v)
            pltpu.sync_copy(
                ids2_hbm.at[pl.ds(pl.multiple_of(cb // L, C // L), C // L)],
                ids_v)
            lo_j = jnp.maximum(done, cb) - cb
            hi_j = jnp.minimum(hi, cb + C) - cb
            hi_j = jnp.maximum(hi_j, lo_j)

            def group_body(t, carry):
                idv = ids_v[t, :]
                for u in range(L):
                    r = t * L + u

                    @pl.when((r >= lo_j) & (r < hi_j))
                    def _row():
                        rx = r + shift
                        xk = [x_v[rx, pl.ds(L * k, L)] for k in range(DK)]
                        acc = xk[0] * wgk[0]
                        for k in range(1, DK):
                            acc = acc + xk[k] * wgk[k]
                        z = jnp.sum(acc)
                        gv = 1.0 / (1.0 + jnp.exp(
                            -(lax.broadcast(z, (L,)) + bg16)))
                        sloc = idv[u] - seg0
                        for k in range(DK):
                            sl = pl.ds(L * k, L)
                            acc_v[sloc, sl] = acc_v[sloc, sl] + xk[k] * gv
                        slg = pl.ds(D, L)
                        acc_v[sloc, slg] = acc_v[sloc, slg] + gv * lane0
                return carry

            lax.fori_loop(lo_j // L, (hi_j + L - 1) // L, group_body, 0)
            return jnp.maximum(done, jnp.minimum(hi, cb + C))

        lax.fori_loop(jlo, jhi, chunk_body, lo)

        pltpu.sync_copy(acc_v, s_out.at[pl.ds(seg0, SPW)])

    return sc_kernel


_SC_KERNEL = _make_sc_kernel()


def _combine_body(s_ref, w_ref, b_ref, o_ref):
    o_ref[...] = jax.lax.dot_general(
        s_ref[:, :D], w_ref[...], (((1,), (0,)), ((), ())),
        preferred_element_type=jnp.float32,
        precision=jax.lax.Precision.HIGHEST) \
        + s_ref[:, D:D + 1] * b_ref[...]


def kernel(x, batch, Wg, bg, W, b):
    ids = batch.astype(jnp.int32)
    ids2 = ids.reshape(NROW16, L)
    wg = Wg.reshape(D)
    bgv = jnp.full((L,), bg[0], dtype=jnp.float32)
    bounds = jnp.searchsorted(
        ids, jnp.arange(0, G + SPW, SPW, dtype=jnp.int32)).astype(jnp.int32)
    bounds = jnp.pad(bounds, (0, 48 - bounds.shape[0]),
                     constant_values=N)

    s_part = _SC_KERNEL(x, ids2, wg, bgv, bounds)

    out = pl.pallas_call(
        _combine_body,
        out_shape=jax.ShapeDtypeStruct((G, D), jnp.float32),
    )(s_part, W, b.reshape(1, D))
    return out
